# baseline (speedup 1.0000x reference)
import jax
import jax.numpy as jnp
from jax.experimental import pallas as pl
from jax.experimental.pallas import tpu as pltpu

_BM = 200


def _body(x_ref, adj_ref, w_ref, b_ref, out_ref, h_ref):
    layer = pl.program_id(0)
    i = pl.program_id(1)

    @pl.when(layer == 0)
    def _():
        acc = jnp.dot(adj_ref[...], x_ref[...], preferred_element_type=jnp.float32)
        h = jnp.dot(acc, w_ref[0], preferred_element_type=jnp.float32) + b_ref[0]
        h_ref[pl.ds(i * _BM, _BM), :] = jnp.maximum(h, 0.0)

    @pl.when(layer == 1)
    def _():
        acc = jnp.dot(adj_ref[...], h_ref[...], preferred_element_type=jnp.float32)
        h = jnp.dot(acc, w_ref[0], preferred_element_type=jnp.float32) + b_ref[0]
        out_ref[...] = jnp.maximum(h, 0.0)


def kernel(x, adj, W1, b1, W2, b2):
    n, f = x.shape
    w = jnp.stack([W1, W2])
    b = jnp.stack([b1.reshape(1, f), b2.reshape(1, f)])
    return pl.pallas_call(
        _body,
        grid=(2, n // _BM),
        in_specs=[
            pl.BlockSpec((n, f), lambda l, i: (0, 0)),
            pl.BlockSpec((_BM, n), lambda l, i: (i, 0)),
            pl.BlockSpec((1, f, f), lambda l, i: (l, 0, 0)),
            pl.BlockSpec((1, 1, f), lambda l, i: (l, 0, 0)),
        ],
        out_specs=pl.BlockSpec((_BM, f), lambda l, i: (i, 0)),
        out_shape=jax.ShapeDtypeStruct((n, f), jnp.float32),
        scratch_shapes=[pltpu.VMEM((n, f), jnp.float32)],
    )(x, adj, w, b)


# speedup vs baseline: 1.0254x; 1.0254x over previous
import jax
import jax.numpy as jnp
from jax.experimental import pallas as pl
from jax.experimental.pallas import tpu as pltpu

_BM = 400


def _body(x_ref, adj_ref, w_ref, b_ref, out_ref, h_ref):
    layer = pl.program_id(0)
    i = pl.program_id(1)

    @pl.when(layer == 0)
    def _():
        acc = jnp.dot(adj_ref[...], x_ref[...], preferred_element_type=jnp.float32)
        h = jnp.dot(acc, w_ref[0], preferred_element_type=jnp.float32) + b_ref[0]
        h_ref[pl.ds(i * _BM, _BM), :] = jnp.maximum(h, 0.0)

    @pl.when(layer == 1)
    def _():
        acc = jnp.dot(adj_ref[...], h_ref[...], preferred_element_type=jnp.float32)
        h = jnp.dot(acc, w_ref[0], preferred_element_type=jnp.float32) + b_ref[0]
        out_ref[...] = jnp.maximum(h, 0.0)


def kernel(x, adj, W1, b1, W2, b2):
    n, f = x.shape
    w = jnp.stack([W1, W2])
    b = jnp.stack([b1.reshape(1, f), b2.reshape(1, f)])
    return pl.pallas_call(
        _body,
        grid=(2, n // _BM),
        in_specs=[
            pl.BlockSpec((n, f), lambda l, i: (0, 0)),
            pl.BlockSpec((_BM, n), lambda l, i: (i, 0)),
            pl.BlockSpec((1, f, f), lambda l, i: (l, 0, 0)),
            pl.BlockSpec((1, 1, f), lambda l, i: (l, 0, 0)),
        ],
        out_specs=pl.BlockSpec((_BM, f), lambda l, i: (i, 0)),
        out_shape=jax.ShapeDtypeStruct((n, f), jnp.float32),
        scratch_shapes=[pltpu.VMEM((n, f), jnp.float32)],
    )(x, adj, w, b)


# speedup vs baseline: 1.0496x; 1.0237x over previous
import jax
import jax.numpy as jnp
from jax.experimental import pallas as pl
from jax.experimental.pallas import tpu as pltpu

_BM = 400


def _body(x_ref, adj_ref, w1_ref, b1_ref, w2_ref, b2_ref, out_ref, h_ref):
    layer = pl.program_id(0)
    i = pl.program_id(1)

    @pl.when(layer == 0)
    def _():
        acc = jnp.dot(adj_ref[...], x_ref[...], preferred_element_type=jnp.float32)
        h = jnp.dot(acc, w1_ref[...], preferred_element_type=jnp.float32) + b1_ref[...]
        h_ref[pl.ds(i * _BM, _BM), :] = jnp.maximum(h, 0.0)

    @pl.when(layer == 1)
    def _():
        acc = jnp.dot(adj_ref[...], h_ref[...], preferred_element_type=jnp.float32)
        h = jnp.dot(acc, w2_ref[...], preferred_element_type=jnp.float32) + b2_ref[...]
        out_ref[...] = jnp.maximum(h, 0.0)


def kernel(x, adj, W1, b1, W2, b2):
    n, f = x.shape
    const = lambda l, i: (0, 0)
    return pl.pallas_call(
        _body,
        grid=(2, n // _BM),
        in_specs=[
            pl.BlockSpec((n, f), const),
            pl.BlockSpec((_BM, n), lambda l, i: (i, 0)),
            pl.BlockSpec((f, f), const),
            pl.BlockSpec((1, f), const),
            pl.BlockSpec((f, f), const),
            pl.BlockSpec((1, f), const),
        ],
        out_specs=pl.BlockSpec((_BM, f), lambda l, i: (i, 0)),
        out_shape=jax.ShapeDtypeStruct((n, f), jnp.float32),
        scratch_shapes=[pltpu.VMEM((n, f), jnp.float32)],
    )(x, adj, W1, b1.reshape(1, f), W2, b2.reshape(1, f))
